# baseline (speedup 1.0000x reference)
import functools

import jax
import jax.numpy as jnp
from jax import lax
from jax.experimental import pallas as pl
from jax.experimental.pallas import tpu as pltpu
from jax.experimental.pallas import tpu_sc as plsc

NUM_WORKERS = 32
CHUNK = 512
NB = 4
K = 2
BB = 2048


@functools.lru_cache(maxsize=None)
def _build_gather(n_total, vocab, dim):
  lanes_per_row = 128 // dim
  seg = CHUNK // lanes_per_row
  per_w = n_total // NUM_WORKERS
  n = per_w // CHUNK
  assert per_w * NUM_WORKERS == n_total and n * CHUNK == per_w
  assert n % NB == 0 and n // NB >= 2 and per_w % BB == 0
  mesh = plsc.VectorSubcoreMesh(core_axis_name="c", subcore_axis_name="s")

  @functools.partial(
      pl.kernel,
      mesh=mesh,
      out_type=jax.ShapeDtypeStruct((n_total // lanes_per_row,
                                     lanes_per_row, dim), jnp.float32),
      compiler_params=pltpu.CompilerParams(use_tc_tiling_on_sc=False),
      scratch_types=(
          [pltpu.VMEM((NB, CHUNK), jnp.int32),
           pltpu.VMEM((NB, CHUNK, dim), jnp.float32)]
          + [pltpu.SemaphoreType.DMA] * (3 * NB)
      ),
  )
  def emb(idx_hbm, table_hbm, out_hbm, idx_v, rows_v, *sems):
    lsem = sems[0:NB]
    gsem = sems[NB:2 * NB]
    ssem = sems[2 * NB:3 * NB]
    wid = lax.axis_index("s") * 2 + lax.axis_index("c")
    base_w = wid * per_w

    def idx_seg(g, b, q):
      p0 = base_w + g * CHUNK
      src = (p0 // BB) * BB + q * (BB // lanes_per_row) + (p0 % BB) // lanes_per_row
      return pltpu.make_async_copy(
          idx_hbm.at[pl.ds(pl.multiple_of(src, seg), seg)],
          idx_v.at[b, pl.ds(q * seg, seg)], lsem[b])

    def gath(b):
      return pltpu.make_async_copy(
          table_hbm.at[idx_v.at[b]], rows_v.at[b], gsem[b])

    def store_seg(g, b, q):
      p0 = base_w + g * CHUNK
      return pltpu.make_async_copy(
          rows_v.at[b, pl.ds(q * seg, seg), :],
          out_hbm.at[pl.ds(p0 // lanes_per_row, seg), q, :], ssem[b])

    def start_idx(g, b):
      for q in range(lanes_per_row):
        idx_seg(g, b, q).start()

    def wait_idx(g, b):
      for q in range(lanes_per_row):
        idx_seg(g, b, q).wait()

    def start_store(g, b):
      for q in range(lanes_per_row):
        store_seg(g, b, q).start()

    def wait_store(g, b):
      for q in range(lanes_per_row):
        store_seg(g, b, q).wait()

    for b in range(NB):
      start_idx(b, b)
    for j in range(K):
      wait_idx(j, j)
      gath(j).start()

    def step(g, b, do_idx_load, do_store_wait):
      gath(b).wait()
      start_store(g, b)
      if do_idx_load:
        start_idx(g + NB, b)
      b2 = (b + K) % NB
      if do_store_wait:
        wait_store(g + K - NB, b2)
      wait_idx(g + K, b2)
      gath(b2).start()

    for b in range(NB):
      step(b, b, True, b + K >= NB)

    def outer(go, carry):
      for b in range(NB):
        step(go * NB + b, b, True, True)
      return carry

    lax.fori_loop(1, n // NB - 1, outer, 0)

    for b in range(NB):
      g = n - NB + b
      gath(b).wait()
      start_store(g, b)
      if g + K < n:
        b2 = (b + K) % NB
        wait_store(g + K - NB, b2)
        wait_idx(g + K, b2)
        gath(b2).start()
    for b in range(NB):
      wait_store(n - NB + b, b)

  return emb


@functools.lru_cache(maxsize=None)
def _build_transpose(hist, batch, dim):
  lanes_per_row = 128 // dim
  packed_rows = BB // lanes_per_row
  nb = batch // BB
  assert nb * BB == batch

  def trans(in_ref, out_ref, eye_ref):
    @pl.when(jnp.logical_and(pl.program_id(0) == 0, pl.program_id(1) == 0))
    def _init():
      r = lax.broadcasted_iota(jnp.int32, (128, 128), 0)
      c = lax.broadcasted_iota(jnp.int32, (128, 128), 1)
      eye_ref[...] = (r == c).astype(jnp.float32)

    x = in_ref[0]
    xt = lax.dot_general(eye_ref[...], x, (((1,), (1,)), ((), ())),
                         preferred_element_type=jnp.float32)
    out_ref[0] = jnp.concatenate(
        [xt[q * dim:(q + 1) * dim] for q in range(lanes_per_row)], axis=1)

  return pl.pallas_call(
      trans,
      grid=(hist, nb),
      in_specs=[pl.BlockSpec((1, packed_rows, 128), lambda h, j: (h, j, 0))],
      out_specs=pl.BlockSpec((1, dim, BB), lambda h, j: (h, 0, j)),
      out_shape=jax.ShapeDtypeStruct((hist, dim, batch), jnp.float32),
      scratch_shapes=[pltpu.VMEM((128, 128), jnp.float32)],
  )


def kernel(i, table):
  b, h = i.shape
  vocab, dim = table.shape
  n_total = b * h
  idx_t = i.T.reshape(n_total)
  tmp = _build_gather(n_total, vocab, dim)(idx_t, table)
  packed = tmp.reshape(h, b * dim // 128, 128)
  out_phys = _build_transpose(h, b, dim)(packed)
  return out_phys.transpose(2, 0, 1)


# speedup vs baseline: 1.6748x; 1.6748x over previous
import functools

import jax
import jax.numpy as jnp
from jax import lax
from jax.experimental import pallas as pl
from jax.experimental.pallas import tpu as pltpu
from jax.experimental.pallas import tpu_sc as plsc

NUM_WORKERS = 32
CHUNK = 512
NB = 4
K = 2
BB = 2048


@functools.lru_cache(maxsize=None)
def _build_gather(n_total, vocab, dim):
  lanes_per_row = 128 // dim
  seg = CHUNK // lanes_per_row
  per_w = n_total // NUM_WORKERS
  n = per_w // CHUNK
  assert per_w * NUM_WORKERS == n_total and n * CHUNK == per_w
  assert n % NB == 0 and n // NB >= 2 and per_w % BB == 0
  mesh = plsc.VectorSubcoreMesh(core_axis_name="c", subcore_axis_name="s")

  @functools.partial(
      pl.kernel,
      mesh=mesh,
      out_type=jax.ShapeDtypeStruct((n_total // lanes_per_row,
                                     lanes_per_row, dim), jnp.float32),
      compiler_params=pltpu.CompilerParams(use_tc_tiling_on_sc=False),
      scratch_types=(
          [pltpu.VMEM((NB, CHUNK), jnp.int32),
           pltpu.VMEM((NB, CHUNK, dim), jnp.float32)]
          + [pltpu.SemaphoreType.DMA] * (3 * NB)
      ),
  )
  def emb(idx_hbm, table_hbm, out_hbm, idx_v, rows_v, *sems):
    lsem = sems[0:NB]
    gsem = sems[NB:2 * NB]
    ssem = sems[2 * NB:3 * NB]
    wid = lax.axis_index("s") * 2 + lax.axis_index("c")
    base_w = wid * per_w

    def idx_seg(g, b, q):
      p0 = base_w + g * CHUNK
      src = (p0 // BB) * BB + q * (BB // lanes_per_row) + (p0 % BB) // lanes_per_row
      return pltpu.make_async_copy(
          idx_hbm.at[pl.ds(pl.multiple_of(src, seg), seg)],
          idx_v.at[b, pl.ds(q * seg, seg)], lsem[b])

    def gath(b):
      return pltpu.make_async_copy(
          table_hbm.at[idx_v.at[b]], rows_v.at[b], gsem[b])

    def store_seg(g, b, q):
      p0 = base_w + g * CHUNK
      return pltpu.make_async_copy(
          rows_v.at[b, pl.ds(q * seg, seg), :],
          out_hbm.at[pl.ds(p0 // lanes_per_row, seg), q, :], ssem[b])

    def start_idx(g, b):
      for q in range(lanes_per_row):
        idx_seg(g, b, q).start()

    def wait_idx(g, b):
      for q in range(lanes_per_row):
        idx_seg(g, b, q).wait()

    def start_store(g, b):
      for q in range(lanes_per_row):
        store_seg(g, b, q).start()

    def wait_store(g, b):
      for q in range(lanes_per_row):
        store_seg(g, b, q).wait()

    for b in range(NB):
      start_idx(b, b)
    for j in range(K):
      wait_idx(j, j)
      gath(j).start()

    def step(g, b, do_idx_load, do_store_wait):
      gath(b).wait()
      start_store(g, b)
      if do_idx_load:
        start_idx(g + NB, b)
      b2 = (b + K) % NB
      if do_store_wait:
        wait_store(g + K - NB, b2)
      wait_idx(g + K, b2)
      gath(b2).start()

    for b in range(NB):
      step(b, b, True, b + K >= NB)

    def outer(go, carry):
      for b in range(NB):
        step(go * NB + b, b, True, True)
      return carry

    lax.fori_loop(1, n // NB - 1, outer, 0)

    for b in range(NB):
      g = n - NB + b
      gath(b).wait()
      start_store(g, b)
      if g + K < n:
        b2 = (b + K) % NB
        wait_store(g + K - NB, b2)
        wait_idx(g + K, b2)
        gath(b2).start()
    for b in range(NB):
      wait_store(n - NB + b, b)

  return emb


@functools.lru_cache(maxsize=None)
def _build_transpose(hist, batch, dim):
  lanes_per_row = 128 // dim
  packed_rows = BB // lanes_per_row
  nb = batch // BB
  assert nb * BB == batch

  packed_rows_h = batch * dim // 128
  sub = BB // lanes_per_row

  def trans(in_ref, out_ref):
    x = in_ref[0]
    xt = x.T
    out_ref[0] = jnp.concatenate(
        [xt[q * dim:(q + 1) * dim, k * sub:(k + 1) * sub]
         for k in range(batch // BB) for q in range(lanes_per_row)], axis=1)

  return pl.pallas_call(
      trans,
      grid=(hist,),
      in_specs=[pl.BlockSpec((1, packed_rows_h, 128), lambda h: (h, 0, 0))],
      out_specs=pl.BlockSpec((1, dim, batch), lambda h: (h, 0, 0)),
      out_shape=jax.ShapeDtypeStruct((hist, dim, batch), jnp.float32),
  )


def kernel(i, table):
  b, h = i.shape
  vocab, dim = table.shape
  n_total = b * h
  idx_t = i.T.reshape(n_total)
  tmp = _build_gather(n_total, vocab, dim)(idx_t, table)
  packed = tmp.reshape(h, b * dim // 128, 128)
  out_phys = _build_transpose(h, b, dim)(packed)
  return out_phys.transpose(2, 0, 1)
